# Initial kernel scaffold; baseline (speedup 1.0000x reference)
import jax
import jax.numpy as jnp
from jax.experimental import pallas as pl


def kernel(atomic_numbers, neighbour_displacements, neighbour_indices, embedding_table, W_emb, W_rad, W_l0, W_l1, W_l2):
    raise NotImplementedError("write your pallas kernel here")


# speedup vs baseline: 11.5441x; 11.5441x over previous
import functools

import jax
import jax.numpy as jnp
from jax import lax
from jax.experimental import pallas as pl
from jax.experimental.pallas import tpu as pltpu
from jax.experimental.pallas import tpu_sc as plsc

N = 10000
E = 160000
RAD = 32
F = 32
CUTOFF = 5.0
GAMMA = (RAD / CUTOFF) ** 2 * 0.1
S3 = 3.0 ** 0.5
HALF = 144


_BN = 1000


def _node_body(z_ref, tab_ref, wemb_ref, emb_ref, embt_ref):
    z = z_ref[...]
    cols = lax.broadcasted_iota(jnp.int32, (1, 128), 1)
    oh = (z == cols).astype(jnp.float32)
    emb = jnp.dot(oh, tab_ref[...], preferred_element_type=jnp.float32)
    embt = jnp.dot(emb, wemb_ref[...], preferred_element_type=jnp.float32)
    emb_ref[...] = emb
    embt_ref[...] = embt


def _node_tables(z2d, tab_pad, w_emb):
    return pl.pallas_call(
        _node_body,
        grid=(N // _BN,),
        in_specs=[
            pl.BlockSpec((_BN, 1), lambda i: (i, 0)),
            pl.BlockSpec((128, RAD), lambda i: (0, 0)),
            pl.BlockSpec((RAD, RAD), lambda i: (0, 0)),
        ],
        out_specs=[
            pl.BlockSpec((_BN, F), lambda i: (i, 0)),
            pl.BlockSpec((_BN, F), lambda i: (i, 0)),
        ],
        out_shape=[
            jax.ShapeDtypeStruct((N, F), jnp.float32),
            jax.ShapeDtypeStruct((N, F), jnp.float32),
        ],
    )(z2d, tab_pad, w_emb)


_GC = 128
_PER_TILE_G = E // 32
_GFULL = _PER_TILE_G // _GC
_GREM = _PER_TILE_G - _GFULL * _GC


def _gather_body(embn_hbm, embt_hbm, idxj_hbm, idxi_hbm, ej_hbm, eti_hbm,
                 idxbuf, rowbuf, idxr, rowr, sem):
    cid = lax.axis_index("c")
    sid = lax.axis_index("s")
    wid = sid * 2 + cid
    base = wid * _PER_TILE_G

    def one(idx_hbm, tab_hbm, out_hbm):
        @pl.loop(0, _GFULL)
        def _(c):
            off = base + c * _GC
            pltpu.sync_copy(idx_hbm.at[pl.ds(off, _GC)], idxbuf)
            pltpu.async_copy(tab_hbm.at[idxbuf], rowbuf, sem).wait()
            pltpu.sync_copy(rowbuf, out_hbm.at[pl.ds(off, _GC)])
        off = base + _GFULL * _GC
        pltpu.sync_copy(idx_hbm.at[pl.ds(off, _GREM)], idxr)
        pltpu.async_copy(tab_hbm.at[idxr], rowr, sem).wait()
        pltpu.sync_copy(rowr, out_hbm.at[pl.ds(off, _GREM)])

    one(idxj_hbm, embn_hbm, ej_hbm)
    one(idxi_hbm, embt_hbm, eti_hbm)


def _edge_gather(emb_nodes, embt_nodes, idx_j, idx_i):
    mesh = plsc.VectorSubcoreMesh(core_axis_name="c", subcore_axis_name="s")
    fn = functools.partial(
        pl.kernel,
        out_type=[
            jax.ShapeDtypeStruct((E, F), jnp.float32),
            jax.ShapeDtypeStruct((E, F), jnp.float32),
        ],
        mesh=mesh,
        scratch_types=[
            pltpu.VMEM((_GC,), jnp.int32),
            pltpu.VMEM((_GC, F), jnp.float32),
            pltpu.VMEM((_GREM,), jnp.int32),
            pltpu.VMEM((_GREM, F), jnp.float32),
            pltpu.SemaphoreType.DMA,
        ],
        compiler_params=pltpu.CompilerParams(use_tc_tiling_on_sc=False),
    )(_gather_body)
    return fn(emb_nodes, embt_nodes, idx_j, idx_i)


_BE = 640


def _edge_body(disp_ref, ej_ref, eti_ref, wrad_ref, wl_ref, p0_ref, p1_ref):
    d = disp_ref[...]
    x = d[:, 0:1]
    y = d[:, 1:2]
    z = d[:, 2:3]
    r2 = x * x + y * y + z * z
    r = jnp.sqrt(r2 + 1e-12)
    inv = 1.0 / (r + 1e-12)
    ux = x * inv
    uy = y * inv
    uz = z * inv

    centers = lax.broadcasted_iota(jnp.int32, (1, RAD), 1).astype(
        jnp.float32) * (CUTOFF / (RAD - 1))
    dr = r - centers
    rbf = jnp.exp(-GAMMA * dr * dr)
    env = 0.5 * (jnp.cos(jnp.pi * jnp.clip(r / CUTOFF, 0.0, 1.0)) + 1.0)

    t = rbf * ej_ref[...]
    coeff = jnp.dot(t, wrad_ref[...], preferred_element_type=jnp.float32) * env
    cs = jnp.dot(coeff, wl_ref[...], preferred_element_type=jnp.float32)
    c0 = cs[:, 0:32]
    c1 = cs[:, 32:64]
    c2 = cs[:, 64:96]
    g = c0 * jax.nn.sigmoid(c0)
    et = eti_ref[...]
    a = c0 * et
    b = c1 * g * et
    c = c2 * g * et

    h4 = (S3 * ux * uy) * c
    p0_ref[...] = jnp.concatenate(
        [a, uy * b, uz * b, ux * b, h4[:, 0:16]], axis=1)
    p1_ref[...] = jnp.concatenate(
        [h4[:, 16:32], (S3 * uy * uz) * c, (1.5 * uz * uz - 0.5) * c,
         (S3 * ux * uz) * c, (0.5 * S3) * (ux * ux - uy * uy) * c], axis=1)


def _edge_dense(disp, e_j, et_i, w_rad, w_l012):
    return pl.pallas_call(
        _edge_body,
        grid=(E // _BE,),
        in_specs=[
            pl.BlockSpec((_BE, 3), lambda i: (i, 0)),
            pl.BlockSpec((_BE, F), lambda i: (i, 0)),
            pl.BlockSpec((_BE, F), lambda i: (i, 0)),
            pl.BlockSpec((RAD, F), lambda i: (0, 0)),
            pl.BlockSpec((RAD, 3 * F), lambda i: (0, 0)),
        ],
        out_specs=[
            pl.BlockSpec((_BE, HALF), lambda i: (i, 0)),
            pl.BlockSpec((_BE, HALF), lambda i: (i, 0)),
        ],
        out_shape=[
            jax.ShapeDtypeStruct((E, HALF), jnp.float32),
            jax.ShapeDtypeStruct((E, HALF), jnp.float32),
        ],
    )(disp, e_j, et_i, w_rad, w_l012)


_SC_CHUNK = 128
_PER_TILE_S = E // 16
_SFULL = _PER_TILE_S // _SC_CHUNK
_SREM = _PER_TILE_S - _SFULL * _SC_CHUNK
_ROWS_PER_TILE = N // 16


def _scatter_body(idx_hbm, p0_hbm, p1_hbm, init0_hbm, init1_hbm,
                  out0_hbm, out1_hbm, idxbuf, valbuf, idxr, valr, acc):
    cid = lax.axis_index("c")
    sid = lax.axis_index("s")
    row0 = sid * _ROWS_PER_TILE

    @pl.when(cid == 0)
    def _():
        pltpu.sync_copy(init0_hbm.at[pl.ds(row0, _ROWS_PER_TILE)],
                        acc.at[pl.ds(row0, _ROWS_PER_TILE)])

    @pl.when(cid == 1)
    def _():
        pltpu.sync_copy(init1_hbm.at[pl.ds(row0, _ROWS_PER_TILE)],
                        acc.at[pl.ds(row0, _ROWS_PER_TILE)])

    plsc.subcore_barrier()

    base = sid * _PER_TILE_S

    @pl.loop(0, _SFULL)
    def _(cix):
        off = base + cix * _SC_CHUNK
        pltpu.sync_copy(idx_hbm.at[pl.ds(off, _SC_CHUNK)], idxbuf)

        @pl.when(cid == 0)
        def _():
            pltpu.sync_copy(p0_hbm.at[pl.ds(off, _SC_CHUNK)], valbuf)

        @pl.when(cid == 1)
        def _():
            pltpu.sync_copy(p1_hbm.at[pl.ds(off, _SC_CHUNK)], valbuf)

        pltpu.sync_copy(valbuf, acc.at[idxbuf], add=True)

    off = base + _SFULL * _SC_CHUNK
    pltpu.sync_copy(idx_hbm.at[pl.ds(off, _SREM)], idxr)

    @pl.when(cid == 0)
    def _():
        pltpu.sync_copy(p0_hbm.at[pl.ds(off, _SREM)], valr)

    @pl.when(cid == 1)
    def _():
        pltpu.sync_copy(p1_hbm.at[pl.ds(off, _SREM)], valr)

    pltpu.sync_copy(valr, acc.at[idxr], add=True)

    plsc.subcore_barrier()

    @pl.when(cid == 0)
    def _():
        pltpu.sync_copy(acc.at[pl.ds(row0, _ROWS_PER_TILE)],
                        out0_hbm.at[pl.ds(row0, _ROWS_PER_TILE)])

    @pl.when(cid == 1)
    def _():
        pltpu.sync_copy(acc.at[pl.ds(row0, _ROWS_PER_TILE)],
                        out1_hbm.at[pl.ds(row0, _ROWS_PER_TILE)])


def _segment_scatter(idx_i, p0, p1, init0, init1):
    mesh = plsc.VectorSubcoreMesh(core_axis_name="c", subcore_axis_name="s")
    fn = functools.partial(
        pl.kernel,
        out_type=[
            jax.ShapeDtypeStruct((N, HALF), jnp.float32),
            jax.ShapeDtypeStruct((N, HALF), jnp.float32),
        ],
        mesh=mesh,
        scratch_types=[
            pltpu.VMEM((_SC_CHUNK,), jnp.int32),
            pltpu.VMEM((_SC_CHUNK, HALF), jnp.float32),
            pltpu.VMEM((_SREM,), jnp.int32),
            pltpu.VMEM((_SREM, HALF), jnp.float32),
            pltpu.VMEM_SHARED((N, HALF), jnp.float32),
        ],
        compiler_params=pltpu.CompilerParams(use_tc_tiling_on_sc=False),
    )(_scatter_body)
    return fn(idx_i, p0, p1, init0, init1)


@jax.jit
def kernel(atomic_numbers, neighbour_displacements, neighbour_indices,
           embedding_table, W_emb, W_rad, W_l0, W_l1, W_l2):
    z2d = atomic_numbers.astype(jnp.int32).reshape(N, 1)
    tab_pad = jnp.zeros((128, RAD), jnp.float32).at[:embedding_table.shape[0]].set(
        embedding_table)
    idx_i = neighbour_indices[0].astype(jnp.int32)
    idx_j = neighbour_indices[1].astype(jnp.int32)

    emb_nodes, embt_nodes = _node_tables(z2d, tab_pad, W_emb)
    e_j, et_i = _edge_gather(emb_nodes, embt_nodes, idx_j, idx_i)

    w_l012 = jnp.concatenate([W_l0, W_l1, W_l2], axis=1)
    p0, p1 = _edge_dense(neighbour_displacements, e_j, et_i, W_rad, w_l012)

    init0 = jnp.concatenate(
        [embt_nodes, jnp.zeros((N, HALF - F), jnp.float32)], axis=1)
    init1 = jnp.zeros((N, HALF), jnp.float32)
    out0, out1 = _segment_scatter(idx_i, p0, p1, init0, init1)

    full = jnp.concatenate([out0, out1], axis=1).reshape(N, 9, F)
    return full[:, None, :, :]


# speedup vs baseline: 23.8519x; 2.0662x over previous
import functools

import numpy as np

import jax
import jax.numpy as jnp
from jax import lax
from jax.experimental import pallas as pl
from jax.experimental.pallas import tpu as pltpu
from jax.experimental.pallas import tpu_sc as plsc

N = 10000
E = 160000
RAD = 32
F = 32
CUTOFF = 5.0
GAMMA = (RAD / CUTOFF) ** 2 * 0.1
S3 = 3.0 ** 0.5
PW = 160
E4 = E // 4


_BN = 1000


def _node_body(z_ref, tab_ref, wemb_ref, emb_ref, embt_ref):
    z = z_ref[...]
    cols = lax.broadcasted_iota(jnp.int32, (1, 128), 1)
    oh = (z == cols).astype(jnp.float32)
    emb = jnp.dot(oh, tab_ref[...], preferred_element_type=jnp.float32)
    embt = jnp.dot(emb, wemb_ref[...], preferred_element_type=jnp.float32)
    emb_ref[...] = emb
    embt_ref[...] = embt


def _node_tables(z2d, tab_pad, w_emb):
    return pl.pallas_call(
        _node_body,
        grid=(N // _BN,),
        in_specs=[
            pl.BlockSpec((_BN, 1), lambda i: (i, 0)),
            pl.BlockSpec((128, RAD), lambda i: (0, 0)),
            pl.BlockSpec((RAD, RAD), lambda i: (0, 0)),
        ],
        out_specs=[
            pl.BlockSpec((_BN, F), lambda i: (i, 0)),
            pl.BlockSpec((_BN, F), lambda i: (i, 0)),
        ],
        out_shape=[
            jax.ShapeDtypeStruct((N, F), jnp.float32),
            jax.ShapeDtypeStruct((N, F), jnp.float32),
        ],
    )(z2d, tab_pad, w_emb)


_GC = 128
_PER_TILE_G = E // 32
_GFULL = _PER_TILE_G // _GC
_GREM = _PER_TILE_G - _GFULL * _GC


def _gather_body(embn_hbm, embt_hbm, idxj_hbm, idxi_hbm, ej_hbm, eti_hbm,
                 idxbuf, rowbuf, idxr, rowr, sem):
    cid = lax.axis_index("c")
    sid = lax.axis_index("s")
    wid = sid * 2 + cid
    base = wid * _PER_TILE_G

    def one(idx_hbm, tab_hbm, out_hbm):
        @pl.loop(0, _GFULL)
        def _(c):
            off = base + c * _GC
            pltpu.sync_copy(idx_hbm.at[pl.ds(off, _GC)], idxbuf)
            pltpu.async_copy(tab_hbm.at[idxbuf], rowbuf, sem).wait()
            pltpu.sync_copy(rowbuf, out_hbm.at[pl.ds(off, _GC)])
        off = base + _GFULL * _GC
        pltpu.sync_copy(idx_hbm.at[pl.ds(off, _GREM)], idxr)
        pltpu.async_copy(tab_hbm.at[idxr], rowr, sem).wait()
        pltpu.sync_copy(rowr, out_hbm.at[pl.ds(off, _GREM)])

    one(idxj_hbm, embn_hbm, ej_hbm)
    one(idxi_hbm, embt_hbm, eti_hbm)


def _edge_gather(emb_nodes, embt_nodes, idx_j, idx_i):
    mesh = plsc.VectorSubcoreMesh(core_axis_name="c", subcore_axis_name="s")
    fn = functools.partial(
        pl.kernel,
        out_type=[
            jax.ShapeDtypeStruct((E, F), jnp.float32),
            jax.ShapeDtypeStruct((E, F), jnp.float32),
        ],
        mesh=mesh,
        scratch_types=[
            pltpu.VMEM((_GC,), jnp.int32),
            pltpu.VMEM((_GC, F), jnp.float32),
            pltpu.VMEM((_GREM,), jnp.int32),
            pltpu.VMEM((_GREM, F), jnp.float32),
            pltpu.SemaphoreType.DMA,
        ],
        compiler_params=pltpu.CompilerParams(use_tc_tiling_on_sc=False),
    )(_gather_body)
    return fn(emb_nodes, embt_nodes, idx_j, idx_i)


_BE = 3200
_R = _BE // 4
_QC = 3 * 128
_PC = 2 * 4 * PW


def _chan_mats():
    deg = [0, 1, 1, 1, 2, 2, 2, 2, 2]
    bq = np.zeros((_QC, _PC), np.float32)
    bc = np.zeros((36, _PC), np.float32)
    for h, ms in ((0, (0, 1, 2, 3, 4)), (1, (5, 6, 7, 8))):
        for ei in range(4):
            for mi, m in enumerate(ms):
                for f in range(F):
                    col = 640 * h + 160 * ei + 32 * mi + f
                    bq[128 * deg[m] + 32 * ei + f, col] = 1.0
                    bc[4 * m + ei, col] = 1.0
    return bq, bc


_BQ_NP, _BC_NP = _chan_mats()


def _edge_body(xs_ref, ys_ref, zs_ref, ej_ref, eti_ref, wrad_ref, wq_ref,
               kr_ref, bq_ref, bc_ref, p0_ref, p1_ref):
    x = xs_ref[...]
    y = ys_ref[...]
    z = zs_ref[...]
    r2 = x * x + y * y + z * z
    r = jnp.sqrt(r2 + 1e-12)
    inv = 1.0 / (r + 1e-12)
    ux = x * inv
    uy = y * inv
    uz = z * inv
    env = 0.5 * (jnp.cos(jnp.pi * jnp.clip(r / CUTOFF, 0.0, 1.0)) + 1.0)

    kr = kr_ref[...]
    rv = jnp.dot(r, kr, preferred_element_type=jnp.float32)
    envv = jnp.dot(env, kr, preferred_element_type=jnp.float32)

    lanes = lax.broadcasted_iota(jnp.int32, (1, 128), 1)
    centers = (lanes % RAD).astype(jnp.float32) * (CUTOFF / (RAD - 1))
    dr = rv - centers
    rbf = jnp.exp(-GAMMA * dr * dr)

    t = rbf * ej_ref[...]
    coeff = jnp.dot(t, wrad_ref[...],
                    preferred_element_type=jnp.float32) * envv
    cq = jnp.dot(coeff, wq_ref[...],
                 preferred_element_type=jnp.float32)
    c0 = cq[:, :128]
    g = c0 * jax.nn.sigmoid(c0)
    et = eti_ref[...]
    get = g * et
    q = cq * jnp.concatenate([et, get, get], axis=1)

    ones4 = jnp.ones_like(x)
    ym36 = jnp.concatenate([
        ones4, uy, uz, ux,
        (S3 * ux) * uy, (S3 * uy) * uz, 1.5 * uz * uz - 0.5,
        (S3 * ux) * uz, (0.5 * S3) * (ux * ux - uy * uy)], axis=1)

    p = (jnp.dot(q, bq_ref[...], preferred_element_type=jnp.float32) *
         jnp.dot(ym36, bc_ref[...], preferred_element_type=jnp.float32))
    p0_ref[...] = p[:, :640]
    p1_ref[...] = p[:, 640:]


def _edge_dense(xs, ys, zs, ej4, eti4, wrad4, wq, kr, bq, bc):
    return pl.pallas_call(
        _edge_body,
        grid=(E4 // _R,),
        in_specs=[
            pl.BlockSpec((_R, 4), lambda i: (i, 0)),
            pl.BlockSpec((_R, 4), lambda i: (i, 0)),
            pl.BlockSpec((_R, 4), lambda i: (i, 0)),
            pl.BlockSpec((_R, 128), lambda i: (i, 0)),
            pl.BlockSpec((_R, 128), lambda i: (i, 0)),
            pl.BlockSpec((128, 128), lambda i: (0, 0)),
            pl.BlockSpec((128, _QC), lambda i: (0, 0)),
            pl.BlockSpec((4, 128), lambda i: (0, 0)),
            pl.BlockSpec((_QC, _PC), lambda i: (0, 0)),
            pl.BlockSpec((36, _PC), lambda i: (0, 0)),
        ],
        out_specs=[
            pl.BlockSpec((_R, 640), lambda i: (i, 0)),
            pl.BlockSpec((_R, 640), lambda i: (i, 0)),
        ],
        out_shape=[
            jax.ShapeDtypeStruct((E4, 640), jnp.float32),
            jax.ShapeDtypeStruct((E4, 640), jnp.float32),
        ],
    )(xs, ys, zs, ej4, eti4, wrad4, wq, kr, bq, bc)


_SC_CHUNK = 128
_PER_TILE_S = E // 16
_SFULL = _PER_TILE_S // _SC_CHUNK
_SREM = _PER_TILE_S - _SFULL * _SC_CHUNK
_ROWS_PER_TILE = N // 16


def _scatter_body(idx_hbm, p0_hbm, p1_hbm, init0_hbm, init1_hbm,
                  out0_hbm, out1_hbm, idxbuf, valbuf, idxr, valr, acc):
    cid = lax.axis_index("c")
    sid = lax.axis_index("s")
    row0 = sid * _ROWS_PER_TILE

    @pl.when(cid == 0)
    def _():
        pltpu.sync_copy(init0_hbm.at[pl.ds(row0, _ROWS_PER_TILE)],
                        acc.at[pl.ds(row0, _ROWS_PER_TILE)])

    @pl.when(cid == 1)
    def _():
        pltpu.sync_copy(init1_hbm.at[pl.ds(row0, _ROWS_PER_TILE)],
                        acc.at[pl.ds(row0, _ROWS_PER_TILE)])

    plsc.subcore_barrier()

    base = sid * _PER_TILE_S

    @pl.loop(0, _SFULL)
    def _(cix):
        off = base + cix * _SC_CHUNK
        pltpu.sync_copy(idx_hbm.at[pl.ds(off, _SC_CHUNK)], idxbuf)

        @pl.when(cid == 0)
        def _():
            pltpu.sync_copy(p0_hbm.at[pl.ds(off, _SC_CHUNK)], valbuf)

        @pl.when(cid == 1)
        def _():
            pltpu.sync_copy(p1_hbm.at[pl.ds(off, _SC_CHUNK)], valbuf)

        pltpu.sync_copy(valbuf, acc.at[idxbuf], add=True)

    off = base + _SFULL * _SC_CHUNK
    pltpu.sync_copy(idx_hbm.at[pl.ds(off, _SREM)], idxr)

    @pl.when(cid == 0)
    def _():
        pltpu.sync_copy(p0_hbm.at[pl.ds(off, _SREM)], valr)

    @pl.when(cid == 1)
    def _():
        pltpu.sync_copy(p1_hbm.at[pl.ds(off, _SREM)], valr)

    pltpu.sync_copy(valr, acc.at[idxr], add=True)

    plsc.subcore_barrier()

    @pl.when(cid == 0)
    def _():
        pltpu.sync_copy(acc.at[pl.ds(row0, _ROWS_PER_TILE)],
                        out0_hbm.at[pl.ds(row0, _ROWS_PER_TILE)])

    @pl.when(cid == 1)
    def _():
        pltpu.sync_copy(acc.at[pl.ds(row0, _ROWS_PER_TILE)],
                        out1_hbm.at[pl.ds(row0, _ROWS_PER_TILE)])


def _segment_scatter(idx_i, p0, p1, init0, init1):
    mesh = plsc.VectorSubcoreMesh(core_axis_name="c", subcore_axis_name="s")
    fn = functools.partial(
        pl.kernel,
        out_type=[
            jax.ShapeDtypeStruct((N, PW), jnp.float32),
            jax.ShapeDtypeStruct((N, PW), jnp.float32),
        ],
        mesh=mesh,
        scratch_types=[
            pltpu.VMEM((_SC_CHUNK,), jnp.int32),
            pltpu.VMEM((_SC_CHUNK, PW), jnp.float32),
            pltpu.VMEM((_SREM,), jnp.int32),
            pltpu.VMEM((_SREM, PW), jnp.float32),
            pltpu.VMEM_SHARED((N, PW), jnp.float32),
        ],
        compiler_params=pltpu.CompilerParams(use_tc_tiling_on_sc=False),
    )(_scatter_body)
    return fn(idx_i, p0, p1, init0, init1)


@jax.jit
def kernel(atomic_numbers, neighbour_displacements, neighbour_indices,
           embedding_table, W_emb, W_rad, W_l0, W_l1, W_l2):
    z2d = atomic_numbers.astype(jnp.int32).reshape(N, 1)
    tab_pad = jnp.zeros((128, RAD), jnp.float32).at[:embedding_table.shape[0]].set(
        embedding_table)
    idx_i = neighbour_indices[0].astype(jnp.int32)
    idx_j = neighbour_indices[1].astype(jnp.int32)

    emb_nodes, embt_nodes = _node_tables(z2d, tab_pad, W_emb)
    e_j, et_i = _edge_gather(emb_nodes, embt_nodes, idx_j, idx_i)

    xs = neighbour_displacements[:, 0].reshape(E4, 4)
    ys = neighbour_displacements[:, 1].reshape(E4, 4)
    zs = neighbour_displacements[:, 2].reshape(E4, 4)
    ej4 = e_j.reshape(E4, 128)
    eti4 = et_i.reshape(E4, 128)
    eye4 = jnp.eye(4, dtype=jnp.float32)
    wrad4 = jnp.kron(eye4, W_rad)
    wq = jnp.concatenate(
        [jnp.kron(eye4, W_l0), jnp.kron(eye4, W_l1), jnp.kron(eye4, W_l2)],
        axis=1)
    kr = jnp.kron(eye4, jnp.ones((1, RAD), jnp.float32))
    bq = jnp.asarray(_BQ_NP)
    bc = jnp.asarray(_BC_NP)

    p0_4, p1_4 = _edge_dense(xs, ys, zs, ej4, eti4, wrad4, wq, kr, bq, bc)
    p0 = p0_4.reshape(E, PW)
    p1 = p1_4.reshape(E, PW)

    init0 = jnp.concatenate(
        [embt_nodes, jnp.zeros((N, PW - F), jnp.float32)], axis=1)
    init1 = jnp.zeros((N, PW), jnp.float32)
    out0, out1 = _segment_scatter(idx_i, p0, p1, init0, init1)

    full = jnp.concatenate([out0, out1[:, :4 * F]], axis=1).reshape(N, 9, F)
    return full[:, None, :, :]


# speedup vs baseline: 25.7716x; 1.0805x over previous
import functools

import numpy as np

import jax
import jax.numpy as jnp
from jax import lax
from jax.experimental import pallas as pl
from jax.experimental.pallas import tpu as pltpu
from jax.experimental.pallas import tpu_sc as plsc

N = 10000
E = 160000
RAD = 32
F = 32
CUTOFF = 5.0
GAMMA = (RAD / CUTOFF) ** 2 * 0.1
S3 = 3.0 ** 0.5
PW = 160
E4 = E // 4


_BN = 1000


def _node_body(z_ref, tab_ref, wemb_ref, emb_ref, embt_ref):
    z = z_ref[...]
    cols = lax.broadcasted_iota(jnp.int32, (1, 128), 1)
    oh = (z == cols).astype(jnp.float32)
    emb = jnp.dot(oh, tab_ref[...], preferred_element_type=jnp.float32)
    embt = jnp.dot(emb, wemb_ref[...], preferred_element_type=jnp.float32)
    emb_ref[...] = emb
    embt_ref[...] = embt


def _node_tables(z2d, tab_pad, w_emb):
    return pl.pallas_call(
        _node_body,
        grid=(N // _BN,),
        in_specs=[
            pl.BlockSpec((_BN, 1), lambda i: (i, 0)),
            pl.BlockSpec((128, RAD), lambda i: (0, 0)),
            pl.BlockSpec((RAD, RAD), lambda i: (0, 0)),
        ],
        out_specs=[
            pl.BlockSpec((_BN, F), lambda i: (i, 0)),
            pl.BlockSpec((_BN, F), lambda i: (i, 0)),
        ],
        out_shape=[
            jax.ShapeDtypeStruct((N, F), jnp.float32),
            jax.ShapeDtypeStruct((N, F), jnp.float32),
        ],
    )(z2d, tab_pad, w_emb)


_GC = 80
_PER_TILE_G = E // 16
_GCH = _PER_TILE_G // _GC


def _gather_body(embn_hbm, embt_hbm, idxj2_hbm, idxi2_hbm, ej_hbm, eti_hbm,
                 idxall, r0, r1, g0, g1, w0, w1):
    cid = lax.axis_index("c")
    sid = lax.axis_index("s")
    base = sid * _PER_TILE_G
    rowb = (r0, r1)
    semg = (g0, g1)
    semw = (w0, w1)

    def run(idx2_hbm, tab_hbm, out_hbm):
        pltpu.sync_copy(idx2_hbm.at[pl.ds(sid * _GCH, _GCH)], idxall)
        for b in range(2):
            pltpu.async_copy(tab_hbm.at[idxall.at[b]], rowb[b], semg[b])

        @pl.loop(2, _GCH - 1, step=2)
        def _(c0):
            for b in range(2):
                pltpu.make_async_copy(tab_hbm.at[idxall.at[b]], rowb[b],
                                      semg[b]).wait()
                off_o = base + (c0 - 2 + b) * _GC
                pltpu.async_copy(rowb[b], out_hbm.at[pl.ds(off_o, _GC)],
                                 semw[b])
            for b in range(2):
                pltpu.make_async_copy(rowb[b], out_hbm.at[pl.ds(0, _GC)],
                                      semw[b]).wait()
                pltpu.async_copy(tab_hbm.at[idxall.at[c0 + b]], rowb[b],
                                 semg[b])

        for b in range(2):
            pltpu.make_async_copy(tab_hbm.at[idxall.at[b]], rowb[b],
                                  semg[b]).wait()
            off_o = base + (_GCH - 3 + b) * _GC
            pltpu.async_copy(rowb[b], out_hbm.at[pl.ds(off_o, _GC)], semw[b])
        for b in range(2):
            pltpu.make_async_copy(rowb[b], out_hbm.at[pl.ds(0, _GC)],
                                  semw[b]).wait()
        pltpu.async_copy(tab_hbm.at[idxall.at[_GCH - 1]], r0, g0).wait()
        pltpu.sync_copy(r0, out_hbm.at[pl.ds(base + (_GCH - 1) * _GC, _GC)])

    @pl.when(cid == 0)
    def _():
        run(idxj2_hbm, embn_hbm, ej_hbm)

    @pl.when(cid == 1)
    def _():
        run(idxi2_hbm, embt_hbm, eti_hbm)


def _edge_gather(emb_nodes, embt_nodes, idx_j2, idx_i2):
    mesh = plsc.VectorSubcoreMesh(core_axis_name="c", subcore_axis_name="s")
    fn = functools.partial(
        pl.kernel,
        out_type=[
            jax.ShapeDtypeStruct((E, F), jnp.float32),
            jax.ShapeDtypeStruct((E, F), jnp.float32),
        ],
        mesh=mesh,
        scratch_types=(
            [pltpu.VMEM((_GCH, _GC), jnp.int32)] +
            [pltpu.VMEM((_GC, F), jnp.float32)] * 2 +
            [pltpu.SemaphoreType.DMA] * 4
        ),
        compiler_params=pltpu.CompilerParams(use_tc_tiling_on_sc=False),
    )(_gather_body)
    return fn(emb_nodes, embt_nodes, idx_j2, idx_i2)


_BE = 3200
_R = _BE // 4
_QC = 3 * 128
_PC = 2 * 4 * PW


def _chan_mats():
    deg = [0, 1, 1, 1, 2, 2, 2, 2, 2]
    bq = np.zeros((_QC, _PC), np.float32)
    bc = np.zeros((36, _PC), np.float32)
    for h, ms in ((0, (0, 1, 2, 3, 4)), (1, (5, 6, 7, 8))):
        for ei in range(4):
            for mi, m in enumerate(ms):
                for f in range(F):
                    col = 640 * h + 160 * ei + 32 * mi + f
                    bq[128 * deg[m] + 32 * ei + f, col] = 1.0
                    bc[4 * m + ei, col] = 1.0
    return bq, bc


_BQ_NP, _BC_NP = _chan_mats()


def _edge_body(xs_ref, ys_ref, zs_ref, ej_ref, eti_ref, wrad_ref, wq_ref,
               kr_ref, bq_ref, bc_ref, p0_ref, p1_ref):
    x = xs_ref[...]
    y = ys_ref[...]
    z = zs_ref[...]
    r2 = x * x + y * y + z * z
    r = jnp.sqrt(r2 + 1e-12)
    inv = 1.0 / (r + 1e-12)
    ux = x * inv
    uy = y * inv
    uz = z * inv
    env = 0.5 * (jnp.cos(jnp.pi * jnp.clip(r / CUTOFF, 0.0, 1.0)) + 1.0)

    kr = kr_ref[...]
    rv = jnp.dot(r, kr, preferred_element_type=jnp.float32)
    envv = jnp.dot(env, kr, preferred_element_type=jnp.float32)

    lanes = lax.broadcasted_iota(jnp.int32, (1, 128), 1)
    centers = (lanes % RAD).astype(jnp.float32) * (CUTOFF / (RAD - 1))
    dr = rv - centers
    rbf = jnp.exp(-GAMMA * dr * dr)

    t = rbf * ej_ref[...]
    coeff = jnp.dot(t, wrad_ref[...],
                    preferred_element_type=jnp.float32) * envv
    cq = jnp.dot(coeff, wq_ref[...],
                 preferred_element_type=jnp.float32)
    c0 = cq[:, :128]
    g = c0 * jax.nn.sigmoid(c0)
    et = eti_ref[...]
    get = g * et
    q = cq * jnp.concatenate([et, get, get], axis=1)

    ones4 = jnp.ones_like(x)
    ym36 = jnp.concatenate([
        ones4, uy, uz, ux,
        (S3 * ux) * uy, (S3 * uy) * uz, 1.5 * uz * uz - 0.5,
        (S3 * ux) * uz, (0.5 * S3) * (ux * ux - uy * uy)], axis=1)

    p = (jnp.dot(q, bq_ref[...], preferred_element_type=jnp.float32) *
         jnp.dot(ym36, bc_ref[...], preferred_element_type=jnp.float32))
    p0_ref[...] = p[:, :640]
    p1_ref[...] = p[:, 640:]


def _edge_dense(xs, ys, zs, ej4, eti4, wrad4, wq, kr, bq, bc):
    return pl.pallas_call(
        _edge_body,
        grid=(E4 // _R,),
        in_specs=[
            pl.BlockSpec((_R, 4), lambda i: (i, 0)),
            pl.BlockSpec((_R, 4), lambda i: (i, 0)),
            pl.BlockSpec((_R, 4), lambda i: (i, 0)),
            pl.BlockSpec((_R, 128), lambda i: (i, 0)),
            pl.BlockSpec((_R, 128), lambda i: (i, 0)),
            pl.BlockSpec((128, 128), lambda i: (0, 0)),
            pl.BlockSpec((128, _QC), lambda i: (0, 0)),
            pl.BlockSpec((4, 128), lambda i: (0, 0)),
            pl.BlockSpec((_QC, _PC), lambda i: (0, 0)),
            pl.BlockSpec((36, _PC), lambda i: (0, 0)),
        ],
        out_specs=[
            pl.BlockSpec((_R, 640), lambda i: (i, 0)),
            pl.BlockSpec((_R, 640), lambda i: (i, 0)),
        ],
        out_shape=[
            jax.ShapeDtypeStruct((E4, 640), jnp.float32),
            jax.ShapeDtypeStruct((E4, 640), jnp.float32),
        ],
    )(xs, ys, zs, ej4, eti4, wrad4, wq, kr, bq, bc)


_SC_CHUNK = 80
_PER_TILE_S = E // 16
_SCH = _PER_TILE_S // _SC_CHUNK
_ROWS_PER_TILE = N // 16


def _scatter_body(idx_hbm, p0_hbm, p1_hbm, zinit_hbm,
                  out0_hbm, out1_hbm,
                  i0, i1, v0, v1, r0, r1, s0, s1, acc):
    cid = lax.axis_index("c")
    sid = lax.axis_index("s")
    row0 = sid * _ROWS_PER_TILE
    idxb = (i0, i1)
    valb = (v0, v1)
    semr = (r0, r1)
    sems = (s0, s1)

    pltpu.sync_copy(zinit_hbm.at[pl.ds(row0, _ROWS_PER_TILE)],
                    acc.at[pl.ds(row0, _ROWS_PER_TILE)])
    plsc.subcore_barrier()

    base = sid * _PER_TILE_S

    def run(p_hbm):
        for b in range(2):
            off = base + b * _SC_CHUNK
            pltpu.sync_copy(idx_hbm.at[pl.ds(off, _SC_CHUNK)], idxb[b])
            pltpu.async_copy(p_hbm.at[pl.ds(off, _SC_CHUNK)], valb[b],
                             semr[b])

        @pl.loop(2, _SCH - 1, step=2)
        def _(c0):
            for b in range(2):
                pltpu.make_async_copy(p_hbm.at[pl.ds(0, _SC_CHUNK)], valb[b],
                                      semr[b]).wait()
                pltpu.async_copy(valb[b], acc.at[idxb[b]], sems[b], add=True)
            for b in range(2):
                pltpu.make_async_copy(valb[b], acc.at[idxb[b]],
                                      sems[b]).wait()
                off_n = base + (c0 + b) * _SC_CHUNK
                pltpu.sync_copy(idx_hbm.at[pl.ds(off_n, _SC_CHUNK)], idxb[b])
                pltpu.async_copy(p_hbm.at[pl.ds(off_n, _SC_CHUNK)], valb[b],
                                 semr[b])

        for b in range(2):
            pltpu.make_async_copy(p_hbm.at[pl.ds(0, _SC_CHUNK)], valb[b],
                                  semr[b]).wait()
            pltpu.async_copy(valb[b], acc.at[idxb[b]], sems[b], add=True)
        for b in range(2):
            pltpu.make_async_copy(valb[b], acc.at[idxb[b]], sems[b]).wait()
        off = base + (_SCH - 1) * _SC_CHUNK
        pltpu.sync_copy(idx_hbm.at[pl.ds(off, _SC_CHUNK)], i0)
        pltpu.sync_copy(p_hbm.at[pl.ds(off, _SC_CHUNK)], v0)
        pltpu.sync_copy(v0, acc.at[i0], add=True)

    @pl.when(cid == 0)
    def _():
        run(p0_hbm)

    @pl.when(cid == 1)
    def _():
        run(p1_hbm)

    plsc.subcore_barrier()

    @pl.when(cid == 0)
    def _():
        pltpu.sync_copy(acc.at[pl.ds(row0, _ROWS_PER_TILE)],
                        out0_hbm.at[pl.ds(row0, _ROWS_PER_TILE)])

    @pl.when(cid == 1)
    def _():
        pltpu.sync_copy(acc.at[pl.ds(row0, _ROWS_PER_TILE)],
                        out1_hbm.at[pl.ds(row0, _ROWS_PER_TILE)])


def _segment_scatter(idx_i, p0, p1, zinit):
    mesh = plsc.VectorSubcoreMesh(core_axis_name="c", subcore_axis_name="s")
    fn = functools.partial(
        pl.kernel,
        out_type=[
            jax.ShapeDtypeStruct((N, PW), jnp.float32),
            jax.ShapeDtypeStruct((N, PW), jnp.float32),
        ],
        mesh=mesh,
        scratch_types=(
            [pltpu.VMEM((_SC_CHUNK,), jnp.int32)] * 2 +
            [pltpu.VMEM((_SC_CHUNK, PW), jnp.float32)] * 2 +
            [pltpu.SemaphoreType.DMA] * 4 +
            [pltpu.VMEM_SHARED((N, PW), jnp.float32)]
        ),
        compiler_params=pltpu.CompilerParams(use_tc_tiling_on_sc=False),
    )(_scatter_body)
    return fn(idx_i, p0, p1, zinit)


@jax.jit
def kernel(atomic_numbers, neighbour_displacements, neighbour_indices,
           embedding_table, W_emb, W_rad, W_l0, W_l1, W_l2):
    z2d = atomic_numbers.astype(jnp.int32).reshape(N, 1)
    tab_pad = jnp.zeros((128, RAD), jnp.float32).at[:embedding_table.shape[0]].set(
        embedding_table)
    idx_i = neighbour_indices[0].astype(jnp.int32)
    idx_j = neighbour_indices[1].astype(jnp.int32)

    emb_nodes, embt_nodes = _node_tables(z2d, tab_pad, W_emb)
    e_j, et_i = _edge_gather(emb_nodes, embt_nodes,
                             idx_j.reshape(E // _GC, _GC),
                             idx_i.reshape(E // _GC, _GC))

    xs = neighbour_displacements[:, 0].reshape(E4, 4)
    ys = neighbour_displacements[:, 1].reshape(E4, 4)
    zs = neighbour_displacements[:, 2].reshape(E4, 4)
    ej4 = e_j.reshape(E4, 128)
    eti4 = et_i.reshape(E4, 128)
    eye4 = jnp.eye(4, dtype=jnp.float32)
    wrad4 = jnp.kron(eye4, W_rad)
    wq = jnp.concatenate(
        [jnp.kron(eye4, W_l0), jnp.kron(eye4, W_l1), jnp.kron(eye4, W_l2)],
        axis=1)
    kr = jnp.kron(eye4, jnp.ones((1, RAD), jnp.float32))
    bq = jnp.asarray(_BQ_NP)
    bc = jnp.asarray(_BC_NP)

    p0_4, p1_4 = _edge_dense(xs, ys, zs, ej4, eti4, wrad4, wq, kr, bq, bc)
    p0 = p0_4.reshape(E, PW)
    p1 = p1_4.reshape(E, PW)

    zinit = jnp.zeros((N, PW), jnp.float32)
    out0, out1 = _segment_scatter(idx_i, p0, p1, zinit)

    out0 = out0.at[:, :F].add(embt_nodes)
    full = jnp.concatenate([out0, out1[:, :4 * F]], axis=1).reshape(N, 9, F)
    return full[:, None, :, :]


# speedup vs baseline: 33.4741x; 1.2989x over previous
import functools

import numpy as np

import jax
import jax.numpy as jnp
from jax import lax
from jax.experimental import pallas as pl
from jax.experimental.pallas import tpu as pltpu
from jax.experimental.pallas import tpu_sc as plsc

N = 10000
E = 160000
RAD = 32
F = 32
CUTOFF = 5.0
GAMMA = (RAD / CUTOFF) ** 2 * 0.1
S3 = 3.0 ** 0.5
PW = 160
E4 = E // 4


_BN = 1000


def _node_body(z_ref, tab_ref, wemb_ref, emb_ref, embt_ref):
    z = z_ref[...]
    cols = lax.broadcasted_iota(jnp.int32, (1, 128), 1)
    oh = (z == cols).astype(jnp.float32)
    emb = jnp.dot(oh, tab_ref[...], preferred_element_type=jnp.float32)
    embt = jnp.dot(emb, wemb_ref[...], preferred_element_type=jnp.float32)
    emb_ref[...] = emb
    embt_ref[...] = embt


def _node_tables(z2d, tab_pad, w_emb):
    return pl.pallas_call(
        _node_body,
        grid=(N // _BN,),
        in_specs=[
            pl.BlockSpec((_BN, 1), lambda i: (i, 0)),
            pl.BlockSpec((128, RAD), lambda i: (0, 0)),
            pl.BlockSpec((RAD, RAD), lambda i: (0, 0)),
        ],
        out_specs=[
            pl.BlockSpec((_BN, F), lambda i: (i, 0)),
            pl.BlockSpec((_BN, F), lambda i: (i, 0)),
        ],
        out_shape=[
            jax.ShapeDtypeStruct((N, F), jnp.float32),
            jax.ShapeDtypeStruct((N, F), jnp.float32),
        ],
    )(z2d, tab_pad, w_emb)


_GC = 80
_PER_TILE_G = E // 16
_GCH = _PER_TILE_G // _GC


def _gather_body(embn_hbm, embt_hbm, idxj2_hbm, idxi2_hbm, ej_hbm, eti_hbm,
                 idxall, r0, r1, g0, g1, w0, w1):
    cid = lax.axis_index("c")
    sid = lax.axis_index("s")
    base = sid * _PER_TILE_G
    rowb = (r0, r1)
    semg = (g0, g1)
    semw = (w0, w1)

    def run(idx2_hbm, tab_hbm, out_hbm):
        pltpu.sync_copy(idx2_hbm.at[pl.ds(sid * _GCH, _GCH)], idxall)
        for b in range(2):
            pltpu.async_copy(tab_hbm.at[idxall.at[b]], rowb[b], semg[b])

        @pl.loop(2, _GCH - 1, step=2)
        def _(c0):
            for b in range(2):
                pltpu.make_async_copy(tab_hbm.at[idxall.at[b]], rowb[b],
                                      semg[b]).wait()
                off_o = base + (c0 - 2 + b) * _GC
                pltpu.async_copy(rowb[b], out_hbm.at[pl.ds(off_o, _GC)],
                                 semw[b])
            for b in range(2):
                pltpu.make_async_copy(rowb[b], out_hbm.at[pl.ds(0, _GC)],
                                      semw[b]).wait()
                pltpu.async_copy(tab_hbm.at[idxall.at[c0 + b]], rowb[b],
                                 semg[b])

        for b in range(2):
            pltpu.make_async_copy(tab_hbm.at[idxall.at[b]], rowb[b],
                                  semg[b]).wait()
            off_o = base + (_GCH - 3 + b) * _GC
            pltpu.async_copy(rowb[b], out_hbm.at[pl.ds(off_o, _GC)], semw[b])
        for b in range(2):
            pltpu.make_async_copy(rowb[b], out_hbm.at[pl.ds(0, _GC)],
                                  semw[b]).wait()
        pltpu.async_copy(tab_hbm.at[idxall.at[_GCH - 1]], r0, g0).wait()
        pltpu.sync_copy(r0, out_hbm.at[pl.ds(base + (_GCH - 1) * _GC, _GC)])

    @pl.when(cid == 0)
    def _():
        run(idxj2_hbm, embn_hbm, ej_hbm)

    @pl.when(cid == 1)
    def _():
        run(idxi2_hbm, embt_hbm, eti_hbm)


def _edge_gather(emb_nodes, embt_nodes, idx_j2, idx_i2):
    mesh = plsc.VectorSubcoreMesh(core_axis_name="c", subcore_axis_name="s")
    fn = functools.partial(
        pl.kernel,
        out_type=[
            jax.ShapeDtypeStruct((E, F), jnp.float32),
            jax.ShapeDtypeStruct((E, F), jnp.float32),
        ],
        mesh=mesh,
        scratch_types=(
            [pltpu.VMEM((_GCH, _GC), jnp.int32)] +
            [pltpu.VMEM((_GC, F), jnp.float32)] * 2 +
            [pltpu.SemaphoreType.DMA] * 4
        ),
        compiler_params=pltpu.CompilerParams(use_tc_tiling_on_sc=False),
    )(_gather_body)
    return fn(emb_nodes, embt_nodes, idx_j2, idx_i2)


_BE = 3200
_R = _BE // 4
_QC = 3 * 128
_PC = 9 * 128


def _chan_mats():
    deg = [0, 1, 1, 1, 2, 2, 2, 2, 2]
    bq = np.zeros((_QC, _PC), np.float32)
    bc = np.zeros((36, _PC), np.float32)
    for m in range(9):
        for ei in range(4):
            for f in range(F):
                col = 128 * m + 32 * ei + f
                bq[128 * deg[m] + 32 * ei + f, col] = 1.0
                bc[4 * m + ei, col] = 1.0
    return bq, bc


_BQ_NP, _BC_NP = _chan_mats()


def _edge_body(xs_ref, ys_ref, zs_ref, ej_ref, eti_ref, wrad_ref, wq_ref,
               kr_ref, bq_ref, bc_ref, *p_refs):
    x = xs_ref[...]
    y = ys_ref[...]
    z = zs_ref[...]
    r2 = x * x + y * y + z * z
    r = jnp.sqrt(r2 + 1e-12)
    inv = 1.0 / (r + 1e-12)
    ux = x * inv
    uy = y * inv
    uz = z * inv
    env = 0.5 * (jnp.cos(jnp.pi * jnp.clip(r / CUTOFF, 0.0, 1.0)) + 1.0)

    kr = kr_ref[...]
    rv = jnp.dot(r, kr, preferred_element_type=jnp.float32)
    envv = jnp.dot(env, kr, preferred_element_type=jnp.float32)

    lanes = lax.broadcasted_iota(jnp.int32, (1, 128), 1)
    centers = (lanes % RAD).astype(jnp.float32) * (CUTOFF / (RAD - 1))
    dr = rv - centers
    rbf = jnp.exp(-GAMMA * dr * dr)

    t = rbf * ej_ref[...]
    coeff = jnp.dot(t, wrad_ref[...],
                    preferred_element_type=jnp.float32) * envv
    cq = jnp.dot(coeff, wq_ref[...],
                 preferred_element_type=jnp.float32)
    c0 = cq[:, :128]
    g = c0 * jax.nn.sigmoid(c0)
    et = eti_ref[...]
    get = g * et
    q = cq * jnp.concatenate([et, get, get], axis=1)

    ones4 = jnp.ones_like(x)
    ym36 = jnp.concatenate([
        ones4, uy, uz, ux,
        (S3 * ux) * uy, (S3 * uy) * uz, 1.5 * uz * uz - 0.5,
        (S3 * ux) * uz, (0.5 * S3) * (ux * ux - uy * uy)], axis=1)

    p = (jnp.dot(q, bq_ref[...], preferred_element_type=jnp.float32) *
         jnp.dot(ym36, bc_ref[...], preferred_element_type=jnp.float32))
    for g in range(9):
        p_refs[g][...] = p[:, 128 * g:128 * (g + 1)]


def _edge_dense(xs, ys, zs, ej4, eti4, wrad4, wq, kr, bq, bc):
    return pl.pallas_call(
        _edge_body,
        grid=(E4 // _R,),
        in_specs=[
            pl.BlockSpec((_R, 4), lambda i: (i, 0)),
            pl.BlockSpec((_R, 4), lambda i: (i, 0)),
            pl.BlockSpec((_R, 4), lambda i: (i, 0)),
            pl.BlockSpec((_R, 128), lambda i: (i, 0)),
            pl.BlockSpec((_R, 128), lambda i: (i, 0)),
            pl.BlockSpec((128, 128), lambda i: (0, 0)),
            pl.BlockSpec((128, _QC), lambda i: (0, 0)),
            pl.BlockSpec((4, 128), lambda i: (0, 0)),
            pl.BlockSpec((_QC, _PC), lambda i: (0, 0)),
            pl.BlockSpec((36, _PC), lambda i: (0, 0)),
        ],
        out_specs=[pl.BlockSpec((_R, 128), lambda i: (i, 0))
                   for _ in range(9)],
        out_shape=[jax.ShapeDtypeStruct((E4, 128), jnp.float32)
                   for _ in range(9)],
    )(xs, ys, zs, ej4, eti4, wrad4, wq, kr, bq, bc)


_SC_CHUNK = 80
_PER_TILE_S = E // 16
_SCH = _PER_TILE_S // _SC_CHUNK
_ROWS_PER_TILE = N // 16
_NCH = 5


def _scatter_body(idx_hbm, p0, p1, p2, p3, p4, p5, p6, p7, p8, zinit_hbm,
                  o0, o1, o2, o3, o4, o5, o6, o7, o8,
                  i0, i1,
                  v00, v01, v02, v03, v04, v10, v11, v12, v13, v14,
                  r0, r1, s0, s1,
                  a0, a1, a2, a3, a4):
    cid = lax.axis_index("c")
    sid = lax.axis_index("s")
    row0 = sid * _ROWS_PER_TILE
    idxb = (i0, i1)
    valb = ((v00, v01, v02, v03, v04), (v10, v11, v12, v13, v14))
    semr = (r0, r1)
    sems = (s0, s1)
    accs = (a0, a1, a2, a3, a4)

    def seed(nch):
        for g in range(nch):
            pltpu.sync_copy(zinit_hbm.at[pl.ds(row0, _ROWS_PER_TILE)],
                            accs[g].at[pl.ds(row0, _ROWS_PER_TILE)])

    base = sid * _PER_TILE_S

    def run(chans):
        nch = len(chans)

        def reads(b, off):
            pltpu.sync_copy(idx_hbm.at[pl.ds(off, _SC_CHUNK)], idxb[b])
            for g in range(nch):
                pltpu.async_copy(chans[g].at[pl.ds(off, _SC_CHUNK)],
                                 valb[b][g], semr[b])

        def drain_reads(b):
            for g in range(nch):
                pltpu.make_async_copy(chans[g].at[pl.ds(0, _SC_CHUNK)],
                                      valb[b][g], semr[b]).wait()

        def scatters(b):
            for g in range(nch):
                pltpu.async_copy(valb[b][g], accs[g].at[idxb[b]], sems[b],
                                 add=True)

        def drain_scatters(b):
            for g in range(nch):
                pltpu.make_async_copy(valb[b][g], accs[g].at[idxb[b]],
                                      sems[b]).wait()

        for b in range(2):
            reads(b, base + b * _SC_CHUNK)

        @pl.loop(2, _SCH - 1, step=2)
        def _(c0):
            for b in range(2):
                drain_reads(b)
                scatters(b)
            for b in range(2):
                drain_scatters(b)
                reads(b, base + (c0 + b) * _SC_CHUNK)

        for b in range(2):
            drain_reads(b)
            scatters(b)
        for b in range(2):
            drain_scatters(b)
        off = base + (_SCH - 1) * _SC_CHUNK
        reads(0, off)
        drain_reads(0)
        scatters(0)
        drain_scatters(0)

    def write(outs):
        for g in range(len(outs)):
            pltpu.sync_copy(accs[g].at[pl.ds(row0, _ROWS_PER_TILE)],
                            outs[g].at[pl.ds(row0, _ROWS_PER_TILE)])

    @pl.when(cid == 0)
    def _():
        seed(5)

    @pl.when(cid == 1)
    def _():
        seed(4)

    plsc.subcore_barrier()

    @pl.when(cid == 0)
    def _():
        run((p0, p1, p2, p3, p4))

    @pl.when(cid == 1)
    def _():
        run((p5, p6, p7, p8))

    plsc.subcore_barrier()

    @pl.when(cid == 0)
    def _():
        write((o0, o1, o2, o3, o4))

    @pl.when(cid == 1)
    def _():
        write((o5, o6, o7, o8))


def _segment_scatter(idx_i, ps, zinit):
    mesh = plsc.VectorSubcoreMesh(core_axis_name="c", subcore_axis_name="s")
    fn = functools.partial(
        pl.kernel,
        out_type=[jax.ShapeDtypeStruct((N, F), jnp.float32)
                  for _ in range(9)],
        mesh=mesh,
        scratch_types=(
            [pltpu.VMEM((_SC_CHUNK,), jnp.int32)] * 2 +
            [pltpu.VMEM((_SC_CHUNK, F), jnp.float32)] * (2 * _NCH) +
            [pltpu.SemaphoreType.DMA] * 4 +
            [pltpu.VMEM_SHARED((N, F), jnp.float32)] * _NCH
        ),
        compiler_params=pltpu.CompilerParams(use_tc_tiling_on_sc=False),
    )(_scatter_body)
    return fn(idx_i, *ps, zinit)


@jax.jit
def kernel(atomic_numbers, neighbour_displacements, neighbour_indices,
           embedding_table, W_emb, W_rad, W_l0, W_l1, W_l2):
    z2d = atomic_numbers.astype(jnp.int32).reshape(N, 1)
    tab_pad = jnp.zeros((128, RAD), jnp.float32).at[:embedding_table.shape[0]].set(
        embedding_table)
    idx_i = neighbour_indices[0].astype(jnp.int32)
    idx_j = neighbour_indices[1].astype(jnp.int32)

    emb_nodes, embt_nodes = _node_tables(z2d, tab_pad, W_emb)
    e_j, et_i = _edge_gather(emb_nodes, embt_nodes,
                             idx_j.reshape(E // _GC, _GC),
                             idx_i.reshape(E // _GC, _GC))

    xs = neighbour_displacements[:, 0].reshape(E4, 4)
    ys = neighbour_displacements[:, 1].reshape(E4, 4)
    zs = neighbour_displacements[:, 2].reshape(E4, 4)
    ej4 = e_j.reshape(E4, 128)
    eti4 = et_i.reshape(E4, 128)
    eye4 = jnp.eye(4, dtype=jnp.float32)
    wrad4 = jnp.kron(eye4, W_rad)
    wq = jnp.concatenate(
        [jnp.kron(eye4, W_l0), jnp.kron(eye4, W_l1), jnp.kron(eye4, W_l2)],
        axis=1)
    kr = jnp.kron(eye4, jnp.ones((1, RAD), jnp.float32))
    bq = jnp.asarray(_BQ_NP)
    bc = jnp.asarray(_BC_NP)

    pouts = _edge_dense(xs, ys, zs, ej4, eti4, wrad4, wq, kr, bq, bc)
    ps = [pg.reshape(E, F) for pg in pouts]

    zinit = jnp.zeros((N, F), jnp.float32)
    outs = _segment_scatter(idx_i, ps, zinit)

    o0 = outs[0] + embt_nodes
    full = jnp.concatenate([o0] + list(outs[1:]), axis=1).reshape(N, 9, F)
    return full[:, None, :, :]
